# baseline (speedup 1.0000x reference)
import jax
import jax.numpy as jnp
from jax import lax
from jax.experimental import pallas as pl
from jax.experimental.pallas import tpu as pltpu
from jax.experimental.pallas import tpu_sc as plsc

D_MODEL = 1024
DELTA = 512
VISIBLE_RANGE = 9.0
NIDX = 9
CSTRIDE = 16
NCOMB = NIDX * CSTRIDE

L = 16
NW = 32
B = 4 * 2048
BPW = B // NW
CHUNK = 32
NCHUNK = BPW // CHUNK
NBUF = 3
NREP = 16


def _pos_kernel(tx_hbm, ty_hbm, xenc_hbm, yenc_hbm, out_hbm, comb_hbm,
                tx_v, ty_v, idx_v, b0, b1, b2,
                s0, s1, s2):
    sid = lax.axis_index("s")
    cid = lax.axis_index("c")
    wid = sid * 2 + cid
    base = wid * BPW
    bufs = (b0, b1, b2)
    sems = (s0, s1, s2)

    pltpu.sync_copy(tx_hbm.at[pl.ds(base, BPW)], tx_v)
    pltpu.sync_copy(ty_hbm.at[pl.ds(base, BPW)], ty_v)

    def ibody(g, carry):
        sl = pl.ds(g * L, L)
        xi = (tx_v[sl] * VISIBLE_RANGE).astype(jnp.int32)
        yi = (ty_v[sl] * VISIBLE_RANGE).astype(jnp.int32)
        idx_v[sl] = xi * CSTRIDE + yi
        return carry

    lax.fori_loop(0, BPW // L, ibody, 0)

    @pl.when(sid < NIDX)
    def _():
        pltpu.sync_copy(xenc_hbm.at[pl.ds(DELTA, CSTRIDE)],
                        b0.at[pl.ds(0, CSTRIDE)])
        pltpu.sync_copy(yenc_hbm.at[pl.ds(DELTA, CSTRIDE)],
                        b1.at[pl.ds(0, CSTRIDE)])

        def jbody(j, carry):
            def cbody(k, carry2):
                cs = pl.ds(k * L, L)
                b2[j, cs] = b0[sid, cs] + b1[j, cs]
                return carry2

            return lax.fori_loop(0, D_MODEL // L, cbody, carry)

        lax.fori_loop(0, CSTRIDE, jbody, 0)
        for rep in range(NREP):
            pltpu.sync_copy(b2.at[pl.ds(0, CSTRIDE)],
                            comb_hbm.at[cid, rep,
                                        pl.ds(sid * CSTRIDE, CSTRIDE)])

    plsc.subcore_barrier()
    rep = sid % NREP

    def gather(c):
        return pltpu.async_copy(
            comb_hbm.at[cid, rep].at[idx_v.at[pl.ds(c * CHUNK, CHUNK)]],
            bufs[c % NBUF], sems[c % NBUF])

    def write(c):
        return pltpu.async_copy(
            bufs[c % NBUF], out_hbm.at[pl.ds(base + c * CHUNK, CHUNK)],
            sems[c % NBUF])

    gwaits = [None] * NCHUNK
    wwaits = [None] * NCHUNK
    for c in range(NBUF):
        gwaits[c] = gather(c)
    for c in range(NCHUNK):
        gwaits[c].wait()
        wwaits[c] = write(c)
        j = c - 2
        if j >= 0 and j + NBUF < NCHUNK:
            wwaits[j].wait()
            wwaits[j] = None
            gwaits[j + NBUF] = gather(j + NBUF)
    for c in range(NCHUNK):
        if wwaits[c] is not None:
            wwaits[c].wait()
            wwaits[c] = None


@jax.jit
def _run(tx, ty, xenc, yenc):
    mesh = plsc.VectorSubcoreMesh(core_axis_name="c", subcore_axis_name="s")
    out, _ = pl.kernel(
        out_type=(
            jax.ShapeDtypeStruct((B, D_MODEL), jnp.float32),
            jax.ShapeDtypeStruct((2, NREP, NCOMB, D_MODEL), jnp.float32),
        ),
        mesh=mesh,
        scratch_types=[
            pltpu.VMEM((BPW,), jnp.float32),
            pltpu.VMEM((BPW,), jnp.float32),
            pltpu.VMEM((BPW,), jnp.int32),
        ] + [pltpu.VMEM((CHUNK, D_MODEL), jnp.float32)] * NBUF
        + [pltpu.SemaphoreType.DMA] * NBUF,
    )(_pos_kernel)(tx, ty, xenc, yenc)
    return out


def kernel(tokens, x_encoding, y_encoding):
    tx = tokens[:, :, 0].reshape(-1)
    ty = tokens[:, :, 1].reshape(-1)
    out = _run(tx, ty, x_encoding, y_encoding)
    return out.reshape(tokens.shape[0], tokens.shape[1], D_MODEL)


# speedup vs baseline: 1.0675x; 1.0675x over previous
import jax
import jax.numpy as jnp
from jax import lax
from jax.experimental import pallas as pl
from jax.experimental.pallas import tpu as pltpu
from jax.experimental.pallas import tpu_sc as plsc

D_MODEL = 1024
DELTA = 512
VISIBLE_RANGE = 9.0
NIDX = 9
CSTRIDE = 16
NCOMB = NIDX * CSTRIDE

L = 16
NW = 32
B = 4 * 2048
BPW = B // NW
CHUNK = 32
NCHUNK = BPW // CHUNK
NBUF = 3
NREP = 4


def _pos_kernel(tx_hbm, ty_hbm, xenc_hbm, yenc_hbm, out_hbm, comb_hbm,
                tx_v, ty_v, idx_v, b0, b1, b2,
                s0, s1, s2):
    sid = lax.axis_index("s")
    cid = lax.axis_index("c")
    wid = sid * 2 + cid
    base = wid * BPW
    bufs = (b0, b1, b2)
    sems = (s0, s1, s2)

    pltpu.sync_copy(tx_hbm.at[pl.ds(base, BPW)], tx_v)
    pltpu.sync_copy(ty_hbm.at[pl.ds(base, BPW)], ty_v)

    def ibody(g, carry):
        sl = pl.ds(g * L, L)
        xi = (tx_v[sl] * VISIBLE_RANGE).astype(jnp.int32)
        yi = (ty_v[sl] * VISIBLE_RANGE).astype(jnp.int32)
        idx_v[sl] = xi * CSTRIDE + yi
        return carry

    lax.fori_loop(0, BPW // L, ibody, 0)

    @pl.when(sid < NIDX)
    def _():
        pltpu.sync_copy(xenc_hbm.at[pl.ds(DELTA, CSTRIDE)],
                        b0.at[pl.ds(0, CSTRIDE)])
        pltpu.sync_copy(yenc_hbm.at[pl.ds(DELTA, CSTRIDE)],
                        b1.at[pl.ds(0, CSTRIDE)])

        def jbody(j, carry):
            def cbody(k, carry2):
                cs = pl.ds(k * L, L)
                b2[j, cs] = b0[sid, cs] + b1[j, cs]
                return carry2

            return lax.fori_loop(0, D_MODEL // L, cbody, carry)

        lax.fori_loop(0, CSTRIDE, jbody, 0)
        for rep in range(NREP):
            pltpu.sync_copy(b2.at[pl.ds(0, CSTRIDE)],
                            comb_hbm.at[cid, rep,
                                        pl.ds(sid * CSTRIDE, CSTRIDE)])

    plsc.subcore_barrier()
    rep = sid % NREP

    def gather(c):
        return pltpu.async_copy(
            comb_hbm.at[cid, rep].at[idx_v.at[pl.ds(c * CHUNK, CHUNK)]],
            bufs[c % NBUF], sems[c % NBUF])

    def write(c):
        return pltpu.async_copy(
            bufs[c % NBUF], out_hbm.at[pl.ds(base + c * CHUNK, CHUNK)],
            sems[c % NBUF])

    gwaits = [None] * NCHUNK
    wwaits = [None] * NCHUNK
    for c in range(NBUF):
        gwaits[c] = gather(c)
    for c in range(NCHUNK):
        gwaits[c].wait()
        wwaits[c] = write(c)
        j = c - 2
        if j >= 0 and j + NBUF < NCHUNK:
            wwaits[j].wait()
            wwaits[j] = None
            gwaits[j + NBUF] = gather(j + NBUF)
    for c in range(NCHUNK):
        if wwaits[c] is not None:
            wwaits[c].wait()
            wwaits[c] = None


@jax.jit
def _run(tx, ty, xenc, yenc):
    mesh = plsc.VectorSubcoreMesh(core_axis_name="c", subcore_axis_name="s")
    out, _ = pl.kernel(
        out_type=(
            jax.ShapeDtypeStruct((B, D_MODEL), jnp.float32),
            jax.ShapeDtypeStruct((2, NREP, NCOMB, D_MODEL), jnp.float32),
        ),
        mesh=mesh,
        scratch_types=[
            pltpu.VMEM((BPW,), jnp.float32),
            pltpu.VMEM((BPW,), jnp.float32),
            pltpu.VMEM((BPW,), jnp.int32),
        ] + [pltpu.VMEM((CHUNK, D_MODEL), jnp.float32)] * NBUF
        + [pltpu.SemaphoreType.DMA] * NBUF,
    )(_pos_kernel)(tx, ty, xenc, yenc)
    return out


def kernel(tokens, x_encoding, y_encoding):
    tx = tokens[:, :, 0].reshape(-1)
    ty = tokens[:, :, 1].reshape(-1)
    out = _run(tx, ty, x_encoding, y_encoding)
    return out.reshape(tokens.shape[0], tokens.shape[1], D_MODEL)
